# baseline (speedup 1.0000x reference)
import functools

import jax
import jax.numpy as jnp
from jax import lax
from jax.experimental import pallas as pl
from jax.experimental.pallas import tpu as pltpu
from jax.experimental.pallas import tpu_sc as plsc

_CHUNK = 128


def _gather_body(n_chunks, b_per_w, nc, cos_hbm, sin_hbm, idx_hbm,
                 cos_out, sin_out, idx_v, cos_v, sin_v, sem):
    wid = lax.axis_index("s") * nc + lax.axis_index("c")
    pltpu.sync_copy(idx_hbm.at[pl.ds(wid * n_chunks, n_chunks)], idx_v)
    copies = []
    for j in range(n_chunks):
        dst = cos_v.at[pl.ds(j * _CHUNK, _CHUNK)]
        copies.append(pltpu.async_copy(cos_hbm.at[idx_v.at[j]], dst, sem))
        dst = sin_v.at[pl.ds(j * _CHUNK, _CHUNK)]
        copies.append(pltpu.async_copy(sin_hbm.at[idx_v.at[j]], dst, sem))
    for c in copies:
        c.wait()
    base = wid * b_per_w
    pltpu.sync_copy(cos_v, cos_out.at[pl.ds(base, b_per_w)])
    pltpu.sync_copy(sin_v, sin_out.at[pl.ds(base, b_per_w)])


def kernel(x, position_ids, cos_cached, sin_cached):
    b, s = position_ids.shape
    n = b * s
    d = cos_cached.shape[1]
    info = plsc.get_sparse_core_info()
    nc, ns = info.num_cores, info.num_subcores
    nw = nc * ns
    b_per_w = n // nw
    n_chunks = b_per_w // _CHUNK

    idx = position_ids.reshape(nw * n_chunks, _CHUNK).astype(jnp.int32)
    mesh = plsc.VectorSubcoreMesh(core_axis_name="c", subcore_axis_name="s")

    run = functools.partial(
        pl.kernel,
        mesh=mesh,
        out_type=[
            jax.ShapeDtypeStruct((n, d), jnp.float32),
            jax.ShapeDtypeStruct((n, d), jnp.float32),
        ],
        scratch_types=[
            pltpu.VMEM((n_chunks, _CHUNK), jnp.int32),
            pltpu.VMEM((b_per_w, d), jnp.float32),
            pltpu.VMEM((b_per_w, d), jnp.float32),
            pltpu.SemaphoreType.DMA,
        ],
        compiler_params=pltpu.CompilerParams(use_tc_tiling_on_sc=False),
    )(functools.partial(_gather_body, n_chunks, b_per_w, nc))

    cos_flat, sin_flat = run(cos_cached, sin_cached, idx)
    cos = cos_flat.reshape(b, s, d).astype(x.dtype)
    sin = sin_flat.reshape(b, s, d).astype(x.dtype)
    return (cos, sin)


# speedup vs baseline: 1.7789x; 1.7789x over previous
import functools

import jax
import jax.numpy as jnp
from jax import lax
from jax.experimental import pallas as pl
from jax.experimental.pallas import tpu as pltpu
from jax.experimental.pallas import tpu_sc as plsc

_L = 16
_PB = 128
_D = 64
_NBLK = 128


def _body(blocks_per_w, nc, cos_hbm, sin_hbm, pos_hbm, cos_out, sin_out,
          pos_v, vb_v, idx_v, cbuf, sbuf, gsem, osem):
    wid = lax.axis_index("s") * nc + lax.axis_index("c")

    def do_block(u):
        b = u // 32
        sb = u % 32
        pltpu.sync_copy(pos_hbm.at[u], pos_v)
        for c in range(_PB // _L):
            p = pos_v[pl.ds(c * _L, _L)]
            vb_v[pl.ds(c * _L, _L)] = ((p >> 7) << 10) + (p & 127)

        def row(i, _):
            off = (i // 8) * (1 << 20) + (i % 8) * 128
            for c in range(_PB // _L):
                vb = vb_v[pl.ds(c * _L, _L)]
                idx_v[pl.ds(i * _PB + c * _L, _L)] = vb + off
            return _

        lax.fori_loop(0, _D, row, None)

        gc = pltpu.async_copy(cos_hbm.at[idx_v], cbuf, gsem)
        gs = pltpu.async_copy(sin_hbm.at[idx_v], sbuf, gsem)
        gc.wait()
        gs.wait()

        outs = []
        for dg in range(8):
            src = cbuf.at[pl.ds(dg * 1024, 1024)]
            outs.append(pltpu.async_copy(src, cos_out.at[b, dg, sb], osem))
            src = sbuf.at[pl.ds(dg * 1024, 1024)]
            outs.append(pltpu.async_copy(src, sin_out.at[b, dg, sb], osem))
        for o in outs:
            o.wait()

    for j in range(blocks_per_w):
        do_block(wid * blocks_per_w + j)


def kernel(x, position_ids, cos_cached, sin_cached):
    b, s = position_ids.shape
    n = b * s
    v, d = cos_cached.shape
    info = plsc.get_sparse_core_info()
    nc, ns = info.num_cores, info.num_subcores
    nw = nc * ns
    blocks_per_w = _NBLK // nw

    def phys1d(t):
        a4 = t.T.reshape(d // 8, 8, v // 128, 128).transpose(0, 2, 1, 3)
        return a4.reshape(v * d)

    cos1d = phys1d(cos_cached)
    sin1d = phys1d(sin_cached)
    pos2d = position_ids.reshape(_NBLK, _PB).astype(jnp.int32)

    mesh = plsc.VectorSubcoreMesh(core_axis_name="c", subcore_axis_name="s")
    run = functools.partial(
        pl.kernel,
        mesh=mesh,
        out_type=[
            jax.ShapeDtypeStruct((b, 8, s // _PB, 8 * _PB), jnp.float32),
            jax.ShapeDtypeStruct((b, 8, s // _PB, 8 * _PB), jnp.float32),
        ],
        scratch_types=[
            pltpu.VMEM((_PB,), jnp.int32),
            pltpu.VMEM((_PB,), jnp.int32),
            pltpu.VMEM((_D * _PB,), jnp.int32),
            pltpu.VMEM((_D * _PB,), jnp.float32),
            pltpu.VMEM((_D * _PB,), jnp.float32),
            pltpu.SemaphoreType.DMA,
            pltpu.SemaphoreType.DMA,
        ],
        compiler_params=pltpu.CompilerParams(use_tc_tiling_on_sc=False),
    )(functools.partial(_body, blocks_per_w, nc))

    cos5, sin5 = run(cos1d, sin1d, pos2d)

    def unview(o5):
        o = o5.reshape(b, 8, s // _PB, 8, _PB).transpose(0, 1, 3, 2, 4)
        return o.reshape(b, d, s).transpose(0, 2, 1)

    return (unview(cos5).astype(x.dtype), unview(sin5).astype(x.dtype))


# speedup vs baseline: 1.8358x; 1.0320x over previous
import functools

import jax
import jax.numpy as jnp
from jax import lax
from jax.experimental import pallas as pl
from jax.experimental.pallas import tpu as pltpu
from jax.experimental.pallas import tpu_sc as plsc

_L = 16
_PB = 128
_D = 64
_NBLK = 128


def _body(blocks_per_w, nc, cos_hbm, sin_hbm, pos_hbm, cos_out, sin_out,
          pos_v, vb_v, idx_v, cbuf, sbuf, gsems, osem):
    wid = lax.axis_index("s") * nc + lax.axis_index("c")
    base_u = wid * blocks_per_w
    nwords = _D * _PB

    pltpu.sync_copy(pos_hbm.at[pl.ds(base_u, blocks_per_w)], pos_v)
    for j in range(blocks_per_w):
        for c in range(_PB // _L):
            p = pos_v[j, pl.ds(c * _L, _L)]
            vb_v[pl.ds(c * _L, _L)] = ((p >> 7) << 10) + (p & 127)

        def row(i, _):
            off = (i // 8) * (1 << 20) + (i % 8) * 128
            for c in range(_PB // _L):
                vb = vb_v[pl.ds(c * _L, _L)]
                idx_v[pl.ds(j * nwords + i * _PB + c * _L, _L)] = vb + off
            return _

        lax.fori_loop(0, _D, row, None)

    gathers = []
    for j in range(blocks_per_w):
        src = idx_v.at[pl.ds(j * nwords, nwords)]
        gathers.append((
            pltpu.async_copy(cos_hbm.at[src], cbuf.at[pl.ds(j * nwords, nwords)], gsems[j]),
            pltpu.async_copy(sin_hbm.at[src], sbuf.at[pl.ds(j * nwords, nwords)], gsems[j]),
        ))

    outs = []
    for j in range(blocks_per_w):
        u = base_u + j
        b = u // 32
        sb = u % 32
        gathers[j][0].wait()
        gathers[j][1].wait()
        for dg in range(8):
            src = cbuf.at[pl.ds(j * nwords + dg * 1024, 1024)]
            outs.append(pltpu.async_copy(src, cos_out.at[b, dg, sb], osem))
            src = sbuf.at[pl.ds(j * nwords + dg * 1024, 1024)]
            outs.append(pltpu.async_copy(src, sin_out.at[b, dg, sb], osem))
    for o in outs:
        o.wait()


def kernel(x, position_ids, cos_cached, sin_cached):
    b, s = position_ids.shape
    n = b * s
    v, d = cos_cached.shape
    info = plsc.get_sparse_core_info()
    nc, ns = info.num_cores, info.num_subcores
    nw = nc * ns
    blocks_per_w = _NBLK // nw

    def phys1d(t):
        a4 = t.T.reshape(d // 8, 8, v // 128, 128).transpose(0, 2, 1, 3)
        return a4.reshape(v * d)

    cos1d = phys1d(cos_cached)
    sin1d = phys1d(sin_cached)
    pos2d = position_ids.reshape(_NBLK, _PB).astype(jnp.int32)

    mesh = plsc.VectorSubcoreMesh(core_axis_name="c", subcore_axis_name="s")
    run = functools.partial(
        pl.kernel,
        mesh=mesh,
        out_type=[
            jax.ShapeDtypeStruct((b, 8, s // _PB, 8 * _PB), jnp.float32),
            jax.ShapeDtypeStruct((b, 8, s // _PB, 8 * _PB), jnp.float32),
        ],
        scratch_types=[
            pltpu.VMEM((blocks_per_w, _PB), jnp.int32),
            pltpu.VMEM((_PB,), jnp.int32),
            pltpu.VMEM((blocks_per_w * _D * _PB,), jnp.int32),
            pltpu.VMEM((blocks_per_w * _D * _PB,), jnp.float32),
            pltpu.VMEM((blocks_per_w * _D * _PB,), jnp.float32),
            [pltpu.SemaphoreType.DMA] * blocks_per_w,
            pltpu.SemaphoreType.DMA,
        ],
        compiler_params=pltpu.CompilerParams(use_tc_tiling_on_sc=False),
    )(functools.partial(_body, blocks_per_w, nc))

    cos5, sin5 = run(cos1d, sin1d, pos2d)

    def unview(o5):
        o = o5.reshape(b, 8, s // _PB, 8, _PB).transpose(0, 1, 3, 2, 4)
        return o.reshape(b, d, s).transpose(0, 2, 1)

    return (unview(cos5).astype(x.dtype), unview(sin5).astype(x.dtype))


# speedup vs baseline: 1.9766x; 1.0767x over previous
import functools

import jax
import jax.numpy as jnp
from jax import lax
from jax.experimental import pallas as pl
from jax.experimental.pallas import tpu as pltpu
from jax.experimental.pallas import tpu_sc as plsc

_L = 16
_PB = 128
_D = 64
_NBLK = 128
_NSPLIT = 2


def _body(blocks_per_w, nc, cos_hbm, sin_hbm, pos_hbm, cos_out, sin_out,
          pos_v, vb_v, idx_v, cbuf, sbuf, gsems, osem):
    wid = lax.axis_index("s") * nc + lax.axis_index("c")
    base_u = wid * blocks_per_w
    nwords = _D * _PB
    half = nwords // _NSPLIT

    def build_and_fire(j):
        u = base_u + j
        b = u // 32
        sb = u % 32
        pltpu.sync_copy(pos_hbm.at[sb, b], pos_v)
        for c in range(_PB // _L):
            p = pos_v[pl.ds(c * _L, _L)]
            vb_v[pl.ds(c * _L, _L)] = ((p >> 7) << 10) + (p & 127)

        def row(i, _):
            off = (i // 8) * (1 << 20) + (i % 8) * 128
            for c in range(_PB // _L):
                vb = vb_v[pl.ds(c * _L, _L)]
                idx_v[pl.ds(j * nwords + i * _PB + c * _L, _L)] = vb + off
            return _

        lax.fori_loop(0, _D, row, None)

        gs = []
        for h in range(_NSPLIT):
            src = idx_v.at[pl.ds(j * nwords + h * half, half)]
            gs.append(pltpu.async_copy(
                cos_hbm.at[src], cbuf.at[pl.ds(j * nwords + h * half, half)],
                gsems[j]))
            gs.append(pltpu.async_copy(
                sin_hbm.at[src], sbuf.at[pl.ds(j * nwords + h * half, half)],
                gsems[j]))
        return gs

    gathers = [build_and_fire(j) for j in range(blocks_per_w)]

    outs = []
    for j in range(blocks_per_w):
        u = base_u + j
        b = u // 32
        sb = u % 32
        for g in gathers[j]:
            g.wait()
        for dg in range(8):
            src = cbuf.at[pl.ds(j * nwords + dg * 1024, 1024)]
            outs.append(pltpu.async_copy(src, cos_out.at[b, dg, sb], osem))
            src = sbuf.at[pl.ds(j * nwords + dg * 1024, 1024)]
            outs.append(pltpu.async_copy(src, sin_out.at[b, dg, sb], osem))
    for o in outs:
        o.wait()


def kernel(x, position_ids, cos_cached, sin_cached):
    b, s = position_ids.shape
    v, d = cos_cached.shape
    info = plsc.get_sparse_core_info()
    nc, ns = info.num_cores, info.num_subcores
    nw = nc * ns
    blocks_per_w = _NBLK // nw

    def phys1d(t):
        a4 = t.T.reshape(d // 8, 8, v // 128, 128).transpose(0, 2, 1, 3)
        return a4.reshape(v * d)

    cos1d = phys1d(cos_cached)
    sin1d = phys1d(sin_cached)
    pos3 = position_ids.astype(jnp.int32).reshape(b, s // _PB, _PB).transpose(1, 0, 2)

    mesh = plsc.VectorSubcoreMesh(core_axis_name="c", subcore_axis_name="s")
    run = functools.partial(
        pl.kernel,
        mesh=mesh,
        out_type=[
            jax.ShapeDtypeStruct((b, 8, s // _PB, 8 * _PB), jnp.float32),
            jax.ShapeDtypeStruct((b, 8, s // _PB, 8 * _PB), jnp.float32),
        ],
        scratch_types=[
            pltpu.VMEM((_PB,), jnp.int32),
            pltpu.VMEM((_PB,), jnp.int32),
            pltpu.VMEM((blocks_per_w * _D * _PB,), jnp.int32),
            pltpu.VMEM((blocks_per_w * _D * _PB,), jnp.float32),
            pltpu.VMEM((blocks_per_w * _D * _PB,), jnp.float32),
            [pltpu.SemaphoreType.DMA] * blocks_per_w,
            pltpu.SemaphoreType.DMA,
        ],
        compiler_params=pltpu.CompilerParams(use_tc_tiling_on_sc=False),
    )(functools.partial(_body, blocks_per_w, nc))

    cos5, sin5 = run(cos1d, sin1d, pos3)

    def unview(o5):
        o = o5.reshape(b, 8, s // _PB, 8, _PB).transpose(0, 1, 3, 2, 4)
        return o.reshape(b, d, s).transpose(0, 2, 1)

    return (unview(cos5).astype(x.dtype), unview(sin5).astype(x.dtype))


# speedup vs baseline: 2.0172x; 1.0205x over previous
import functools

import jax
import jax.numpy as jnp
from jax import lax
from jax.experimental import pallas as pl
from jax.experimental.pallas import tpu as pltpu
from jax.experimental.pallas import tpu_sc as plsc

_L = 16
_PB = 128
_D = 64
_NBLK = 128
_NSPLIT = 4


def _body(blocks_per_w, nc, cos_hbm, sin_hbm, pos_hbm, cos_out, sin_out,
          pos_v, vb_v, idx_v, cbuf, sbuf, gsems, osem):
    wid = lax.axis_index("s") * nc + lax.axis_index("c")
    base_u = wid * blocks_per_w
    nwords = _D * _PB
    half = nwords // _NSPLIT

    def build_and_fire(j):
        u = base_u + j
        b = u // 32
        sb = u % 32
        pltpu.sync_copy(pos_hbm.at[sb, b], pos_v)
        for c in range(_PB // _L):
            p = pos_v[pl.ds(c * _L, _L)]
            vb_v[pl.ds(c * _L, _L)] = ((p >> 7) << 10) + (p & 127)

        def row(i, _):
            off = (i // 8) * (1 << 20) + (i % 8) * 128
            for c in range(_PB // _L):
                vb = vb_v[pl.ds(c * _L, _L)]
                idx_v[pl.ds(j * nwords + i * _PB + c * _L, _L)] = vb + off
            return _

        lax.fori_loop(0, _D, row, None)

        gs = []
        for h in range(_NSPLIT):
            src = idx_v.at[pl.ds(j * nwords + h * half, half)]
            gs.append(pltpu.async_copy(
                cos_hbm.at[src], cbuf.at[pl.ds(j * nwords + h * half, half)],
                gsems[j]))
            gs.append(pltpu.async_copy(
                sin_hbm.at[src], sbuf.at[pl.ds(j * nwords + h * half, half)],
                gsems[j]))
        return gs

    gathers = [build_and_fire(j) for j in range(blocks_per_w)]

    outs = []
    for j in range(blocks_per_w):
        u = base_u + j
        b = u // 32
        sb = u % 32
        for g in gathers[j]:
            g.wait()
        for dg in range(8):
            src = cbuf.at[pl.ds(j * nwords + dg * 1024, 1024)]
            outs.append(pltpu.async_copy(src, cos_out.at[b, dg, sb], osem))
            src = sbuf.at[pl.ds(j * nwords + dg * 1024, 1024)]
            outs.append(pltpu.async_copy(src, sin_out.at[b, dg, sb], osem))
    for o in outs:
        o.wait()


def kernel(x, position_ids, cos_cached, sin_cached):
    b, s = position_ids.shape
    v, d = cos_cached.shape
    info = plsc.get_sparse_core_info()
    nc, ns = info.num_cores, info.num_subcores
    nw = nc * ns
    blocks_per_w = _NBLK // nw

    def phys1d(t):
        a4 = t.T.reshape(d // 8, 8, v // 128, 128).transpose(0, 2, 1, 3)
        return a4.reshape(v * d)

    cos1d = phys1d(cos_cached)
    sin1d = phys1d(sin_cached)
    pos3 = position_ids.astype(jnp.int32).reshape(b, s // _PB, _PB).transpose(1, 0, 2)

    mesh = plsc.VectorSubcoreMesh(core_axis_name="c", subcore_axis_name="s")
    run = functools.partial(
        pl.kernel,
        mesh=mesh,
        out_type=[
            jax.ShapeDtypeStruct((b, 8, s // _PB, 8 * _PB), jnp.float32),
            jax.ShapeDtypeStruct((b, 8, s // _PB, 8 * _PB), jnp.float32),
        ],
        scratch_types=[
            pltpu.VMEM((_PB,), jnp.int32),
            pltpu.VMEM((_PB,), jnp.int32),
            pltpu.VMEM((blocks_per_w * _D * _PB,), jnp.int32),
            pltpu.VMEM((blocks_per_w * _D * _PB,), jnp.float32),
            pltpu.VMEM((blocks_per_w * _D * _PB,), jnp.float32),
            [pltpu.SemaphoreType.DMA] * blocks_per_w,
            pltpu.SemaphoreType.DMA,
        ],
        compiler_params=pltpu.CompilerParams(use_tc_tiling_on_sc=False),
    )(functools.partial(_body, blocks_per_w, nc))

    cos5, sin5 = run(cos1d, sin1d, pos3)

    def unview(o5):
        o = o5.reshape(b, 8, s // _PB, 8, _PB).transpose(0, 1, 3, 2, 4)
        return o.reshape(b, d, s).transpose(0, 2, 1)

    return (unview(cos5).astype(x.dtype), unview(sin5).astype(x.dtype))
